# baseline (speedup 1.0000x reference)
import jax
import jax.numpy as jnp
from jax.experimental import pallas as pl

_NS = 18
_B = 4096
_D = 256
_O = 6
_BLK = 1024

def _gumbel_noise():
    gs = []
    for op_i in range(2):
        base = jax.random.fold_in(jax.random.key(42), op_i)
        keys = jax.random.split(base, _NS)
        g = jnp.stack(
            [jax.random.gumbel(keys[i], (_B, _O), jnp.float32)
             for i in range(_NS)])
        gs.append(jnp.transpose(g, (2, 0, 1)))
    return jnp.concatenate(gs, axis=0).reshape(2 * _O * _NS, _B)


def _fwd_kernel(x_ref, xh_ref, g_ref, w1_ref, b1_ref, w2_ref, b2_ref,
                w_ref, b_ref, out_ref):
    blk = x_ref.shape[0]
    dn = (((1,), (1,)), ((), ()))
    wfull = w_ref[...]
    wx = wfull[:, :_D]
    xb = x_ref[...]
    xh = xh_ref[...]

    idxs = []
    tvecs = []
    for o, (wr, br) in enumerate(((w1_ref, b1_ref), (w2_ref, b2_ref))):
        wo = wr[...]
        bo = br[...]
        z = jax.lax.dot_general(wo, xb, dn,
                                preferred_element_type=jnp.float32) + bo
        m = jnp.max(z, axis=0, keepdims=True)
        e = jnp.exp(z - m)
        dist = e / jnp.sum(e, axis=0, keepdims=True)
        logits = jnp.log(dist)

        zh = jax.lax.dot_general(wo, xh, dn,
                                 preferred_element_type=jnp.float32) + bo
        mh = jnp.max(zh, axis=0, keepdims=True)
        eh = jnp.exp(zh - mh)
        disth = eh / jnp.sum(eh, axis=0, keepdims=True)
        tvecs.append(disth[0, :])

        best = None
        idx = None
        for c in range(6):
            r = (o * 6 + c) * _NS
            val = logits[c:c + 1, :] + g_ref[r:r + _NS, :]
            if c == 0:
                best = val
                idx = jnp.zeros_like(val)
            else:
                pred = val > best
                best = jnp.where(pred, val, best)
                idx = jnp.where(pred, jnp.float32(c), idx)
        idxs.append(idx)

    y0 = jax.lax.dot_general(wx, xb, dn,
                             preferred_element_type=jnp.float32) + b_ref[...]

    accs = [jnp.broadcast_to(y0[j:j + 1, :], (_NS, blk)) for j in range(6)]
    ps = []
    for o in range(2):
        p = None
        for c in range(6):
            mf = (idxs[o] == jnp.float32(c)).astype(jnp.float32)
            tc = tvecs[o][c]
            p = mf * tc if p is None else p + mf * tc
            for j in range(6):
                accs[j] = accs[j] + mf * wfull[j, _D + 6 * o + c]
        ps.append(p)

    m = accs[0]
    for j in range(1, 6):
        m = jnp.maximum(m, accs[j])
    es = [jnp.exp(a - m) for a in accs]
    se = es[0]
    for j in range(1, 6):
        se = se + es[j]

    w = ps[0] * ps[1]
    u = w / se
    norm = jnp.sum(w, axis=0, keepdims=True)
    rows = [jnp.sum(u * es[j], axis=0, keepdims=True) / norm
            for j in range(6)]
    out_ref[...] = jnp.concatenate(rows, axis=0).T


def kernel(x, W_opp1, b_opp1, W_opp2, b_opp2, W, b):
    G = _gumbel_noise()
    b1 = b_opp1.reshape(_O, 1)
    b2 = b_opp2.reshape(_O, 1)
    br = b.reshape(_O, 1)

    out = pl.pallas_call(
        _fwd_kernel,
        grid=(_B // _BLK,),
        in_specs=[
            pl.BlockSpec((_BLK, _D), lambda i: (i, 0)),
            pl.BlockSpec((8, _D), lambda i: (0, 0)),
            pl.BlockSpec((2 * _O * _NS, _BLK), lambda i: (0, i)),
            pl.BlockSpec((_O, _D), lambda i: (0, 0)),
            pl.BlockSpec((_O, 1), lambda i: (0, 0)),
            pl.BlockSpec((_O, _D), lambda i: (0, 0)),
            pl.BlockSpec((_O, 1), lambda i: (0, 0)),
            pl.BlockSpec((_O, _D + 2 * _O), lambda i: (0, 0)),
            pl.BlockSpec((_O, 1), lambda i: (0, 0)),
        ],
        out_specs=pl.BlockSpec((_BLK, _O), lambda i: (i, 0)),
        out_shape=jax.ShapeDtypeStruct((_B, _O), jnp.float32),
    )(x, x, G, W_opp1, b1, W_opp2, b2, W, br)
    return out


# speedup vs baseline: 3.8493x; 3.8493x over previous
import jax
import jax.numpy as jnp
from jax.experimental import pallas as pl

_NS = 18
_B = 4096
_D = 256
_O = 6
_BLK = 1024

def _gumbel_noise():
    keys = jnp.concatenate(
        [jax.random.split(jax.random.fold_in(jax.random.key(42), op_i), _NS)
         for op_i in range(2)])
    g = jax.vmap(lambda k: jax.random.gumbel(k, (_B, _O), jnp.float32))(keys)
    g = g.reshape(2, _NS, _B, _O)
    return jnp.transpose(g, (0, 3, 1, 2)).reshape(2 * _O * _NS, _B)


def _fwd_kernel(x_ref, xh_ref, g_ref, w1_ref, b1_ref, w2_ref, b2_ref,
                w_ref, b_ref, out_ref):
    blk = x_ref.shape[0]
    dn = (((1,), (1,)), ((), ()))
    wfull = w_ref[...]
    wx = wfull[:, :_D]
    xb = x_ref[...]
    xh = xh_ref[...]

    idxs = []
    tvecs = []
    for o, (wr, br) in enumerate(((w1_ref, b1_ref), (w2_ref, b2_ref))):
        wo = wr[...]
        bo = br[...]
        z = jax.lax.dot_general(wo, xb, dn,
                                preferred_element_type=jnp.float32) + bo
        m = jnp.max(z, axis=0, keepdims=True)
        e = jnp.exp(z - m)
        dist = e / jnp.sum(e, axis=0, keepdims=True)
        logits = jnp.log(dist)

        zh = jax.lax.dot_general(wo, xh, dn,
                                 preferred_element_type=jnp.float32) + bo
        mh = jnp.max(zh, axis=0, keepdims=True)
        eh = jnp.exp(zh - mh)
        disth = eh / jnp.sum(eh, axis=0, keepdims=True)
        tvecs.append(disth[0, :])

        best = None
        idx = None
        for c in range(6):
            r = (o * 6 + c) * _NS
            val = logits[c:c + 1, :] + g_ref[r:r + _NS, :]
            if c == 0:
                best = val
                idx = jnp.zeros_like(val)
            else:
                pred = val > best
                best = jnp.where(pred, val, best)
                idx = jnp.where(pred, jnp.float32(c), idx)
        idxs.append(idx)

    y0 = jax.lax.dot_general(wx, xb, dn,
                             preferred_element_type=jnp.float32) + b_ref[...]

    accs = [jnp.broadcast_to(y0[j:j + 1, :], (_NS, blk)) for j in range(6)]
    ps = []
    for o in range(2):
        p = None
        for c in range(6):
            mf = (idxs[o] == jnp.float32(c)).astype(jnp.float32)
            tc = tvecs[o][c]
            p = mf * tc if p is None else p + mf * tc
            for j in range(6):
                accs[j] = accs[j] + mf * wfull[j, _D + 6 * o + c]
        ps.append(p)

    m = accs[0]
    for j in range(1, 6):
        m = jnp.maximum(m, accs[j])
    es = [jnp.exp(a - m) for a in accs]
    se = es[0]
    for j in range(1, 6):
        se = se + es[j]

    w = ps[0] * ps[1]
    u = w / se
    norm = jnp.sum(w, axis=0, keepdims=True)
    rows = [jnp.sum(u * es[j], axis=0, keepdims=True) / norm
            for j in range(6)]
    out_ref[...] = jnp.concatenate(rows, axis=0).T


def kernel(x, W_opp1, b_opp1, W_opp2, b_opp2, W, b):
    G = _gumbel_noise()
    b1 = b_opp1.reshape(_O, 1)
    b2 = b_opp2.reshape(_O, 1)
    br = b.reshape(_O, 1)

    out = pl.pallas_call(
        _fwd_kernel,
        grid=(_B // _BLK,),
        in_specs=[
            pl.BlockSpec((_BLK, _D), lambda i: (i, 0)),
            pl.BlockSpec((8, _D), lambda i: (0, 0)),
            pl.BlockSpec((2 * _O * _NS, _BLK), lambda i: (0, i)),
            pl.BlockSpec((_O, _D), lambda i: (0, 0)),
            pl.BlockSpec((_O, 1), lambda i: (0, 0)),
            pl.BlockSpec((_O, _D), lambda i: (0, 0)),
            pl.BlockSpec((_O, 1), lambda i: (0, 0)),
            pl.BlockSpec((_O, _D + 2 * _O), lambda i: (0, 0)),
            pl.BlockSpec((_O, 1), lambda i: (0, 0)),
        ],
        out_specs=pl.BlockSpec((_BLK, _O), lambda i: (i, 0)),
        out_shape=jax.ShapeDtypeStruct((_B, _O), jnp.float32),
    )(x, x, G, W_opp1, b1, W_opp2, b2, W, br)
    return out
